# baseline (speedup 1.0000x reference)
import functools

import jax
import jax.numpy as jnp
from jax import lax
from jax.experimental import pallas as pl
from jax.experimental.pallas import tpu as pltpu
from jax.experimental.pallas import tpu_sc as plsc

B = 4096
HIST = 50
D = 64
L = 16
NC = 2
NS = 16
NW = NC * NS
RPW = B // NW
IDXW = 128
NBUF = 8
GLEN = 56
VPR = D // L
INV = 1.0 / HIST

_mesh = plsc.VectorSubcoreMesh(core_axis_name="c", subcore_axis_name="s")


@functools.partial(
    pl.kernel,
    out_type=jax.ShapeDtypeStruct((B * D,), jnp.float32),
    mesh=_mesh,
    compiler_params=pltpu.CompilerParams(use_tc_tiling_on_sc=False),
    scratch_types=[
        pltpu.VMEM((RPW, IDXW), jnp.int32),
        *[pltpu.VMEM((GLEN, D), jnp.float32) for _ in range(NBUF)],
        pltpu.VMEM((RPW * D,), jnp.float32),
        *[pltpu.SemaphoreType.DMA for _ in range(NBUF)],
    ],
)
def _emb_mean(tok_hbm, table_hbm, out_hbm, idx_v,
              rb0, rb1, rb2, rb3, rb4, rb5, rb6, rb7,
              out_v, sm0, sm1, sm2, sm3, sm4, sm5, sm6, sm7):
    bufs = (rb0, rb1, rb2, rb3, rb4, rb5, rb6, rb7)
    sems = (sm0, sm1, sm2, sm3, sm4, sm5, sm6, sm7)
    wid = lax.axis_index("s") * NC + lax.axis_index("c")

    pltpu.sync_copy(tok_hbm.at[pl.ds(wid * RPW, RPW), :], idx_v)

    for b in range(NBUF):
        pltpu.make_async_copy(
            table_hbm.at[idx_v.at[b, pl.ds(0, GLEN)]], bufs[b], sems[b]).start()

    @pl.loop(0, RPW, step=NBUF)
    def _(g0):
        for b in range(NBUF):
            g = g0 + b
            buf, sem = bufs[b], sems[b]
            pltpu.make_async_copy(
                table_hbm.at[idx_v.at[g, pl.ds(0, GLEN)]], buf, sem).wait()

            def body(j, acc, _buf=buf):
                row = _buf.at[j]
                return tuple(acc[c] + row[pl.ds(c * L, L)]
                             for c in range(VPR))

            acc = lax.fori_loop(
                0, HIST, body,
                tuple(jnp.zeros((L,), jnp.float32) for _ in range(VPR)),
                unroll=10)
            obase = g * D
            for c in range(VPR):
                out_v[pl.ds(obase + c * L, L)] = acc[c] * INV
            nxt = g + NBUF

            @pl.when(nxt < RPW)
            def _():
                pltpu.make_async_copy(
                    table_hbm.at[idx_v.at[nxt, pl.ds(0, GLEN)]],
                    buf, sem).start()

    pltpu.sync_copy(out_v, out_hbm.at[pl.ds(wid * RPW * D, RPW * D)])


def kernel(token_ids, embedding_matrix):
    tok_p = jnp.pad(token_ids, ((0, 0), (0, IDXW - HIST)))
    out = _emb_mean(tok_p, embedding_matrix)
    return out.reshape(B, D)


# speedup vs baseline: 5.5387x; 5.5387x over previous
import functools

import jax
import jax.numpy as jnp
from jax import lax
from jax.experimental import pallas as pl
from jax.experimental.pallas import tpu as pltpu
from jax.experimental.pallas import tpu_sc as plsc

B = 4096
HIST = 50
D = 64
L = 16
NC = 2
NS = 16
NW = NC * NS
RPW = B // NW
NBUF = 8
VPR = D // L
INV = 1.0 / HIST

_mesh = plsc.VectorSubcoreMesh(core_axis_name="c", subcore_axis_name="s")


@functools.partial(
    pl.kernel,
    out_type=jax.ShapeDtypeStruct((B * D,), jnp.float32),
    mesh=_mesh,
    compiler_params=pltpu.CompilerParams(use_tc_tiling_on_sc=False),
    scratch_types=[
        pltpu.VMEM((RPW, HIST), jnp.int32),
        *[pltpu.VMEM((HIST, D), jnp.float32) for _ in range(NBUF)],
        pltpu.VMEM((RPW * D,), jnp.float32),
        *[pltpu.SemaphoreType.DMA for _ in range(NBUF)],
    ],
)
def _emb_mean(tok_hbm, table_hbm, out_hbm, idx_v,
              rb0, rb1, rb2, rb3, rb4, rb5, rb6, rb7,
              out_v, sm0, sm1, sm2, sm3, sm4, sm5, sm6, sm7):
    bufs = (rb0, rb1, rb2, rb3, rb4, rb5, rb6, rb7)
    sems = (sm0, sm1, sm2, sm3, sm4, sm5, sm6, sm7)
    wid = lax.axis_index("s") * NC + lax.axis_index("c")

    pltpu.sync_copy(tok_hbm.at[pl.ds(wid * RPW, RPW), :], idx_v)

    for b in range(NBUF):
        pltpu.make_async_copy(
            table_hbm.at[idx_v.at[b]], bufs[b], sems[b]).start()

    @pl.loop(0, RPW, step=NBUF)
    def _(g0):
        for b in range(NBUF):
            g = g0 + b
            buf, sem = bufs[b], sems[b]
            pltpu.make_async_copy(
                table_hbm.at[idx_v.at[g]], buf, sem).wait()

            def body(j, acc, _buf=buf):
                row = _buf.at[j]
                return tuple(acc[c] + row[pl.ds(c * L, L)]
                             for c in range(VPR))

            acc = lax.fori_loop(
                0, HIST, body,
                tuple(jnp.zeros((L,), jnp.float32) for _ in range(VPR)),
                unroll=10)
            obase = g * D
            for c in range(VPR):
                out_v[pl.ds(obase + c * L, L)] = acc[c] * INV
            nxt = g + NBUF

            @pl.when(nxt < RPW)
            def _():
                pltpu.make_async_copy(
                    table_hbm.at[idx_v.at[nxt]],
                    buf, sem).start()

    pltpu.sync_copy(out_v, out_hbm.at[pl.ds(wid * RPW * D, RPW * D)])


def kernel(token_ids, embedding_matrix):
    out = _emb_mean(token_ids, embedding_matrix)
    return out.reshape(B, D)
